# Initial kernel scaffold; baseline (speedup 1.0000x reference)
import jax
import jax.numpy as jnp
from jax.experimental import pallas as pl


def kernel(s, v, p, e, batch, edge_index_global, W_shared, b_shared, W_bond, b_bond, W_b0, b_b0, W_b1, b_b1, W_coords, W_atoms, b_atoms):
    raise NotImplementedError("write your pallas kernel here")


# speedup vs baseline: 2.6202x; 2.6202x over previous
import functools

import jax
import jax.numpy as jnp
from jax import lax
from jax.experimental import pallas as pl
from jax.experimental.pallas import tpu as pltpu
from jax.experimental.pallas import tpu_sc as plsc

_N = 10000
_E = 320000
_SDIM = 128
_VDIM = 64
_EDGE_DIM = 16
_NUM_ATOM = 16
_NUM_BOND = 5
_G = 256

_NC = 2
_NS = 16
_NW = _NC * _NS
_ET = _E // _NW
_CH = 128
_NFULL = _ET // _CH
_REM = _ET - _NFULL * _CH
_NCHUNK = _NFULL + 1

_TDIM = 144
_BLK = 2560
_BN = 2000
_NB = _N // _BN


def _nodeA_body(s_ref, v2_ref, p_ref, batch_ref, Ws_ref, bs_ref, Wa_ref,
                ba_ref, W0p_ref, bb0_ref, Wbond_ref, bbond_ref, Sw_ref,
                atoms_ref, t_ref, center_ref, sums_ref, wcomb_ref):
    blk = pl.program_id(0)
    s2 = jnp.dot(s_ref[...], Ws_ref[...].T, preferred_element_type=jnp.float32)
    s2 = s2 + bs_ref[...]
    s2 = s2 * jax.nn.sigmoid(s2)

    atoms_ref[...] = (
        jnp.dot(s2, Wa_ref[...].T, preferred_element_type=jnp.float32)
        + ba_ref[...]
    )

    tb = 0.5 * (bb0_ref[...]
                + jnp.dot(bbond_ref[...], W0p_ref[...].T,
                          preferred_element_type=jnp.float32))
    t_ref[...] = (
        jnp.dot(s2, W0p_ref[...].T, preferred_element_type=jnp.float32) + tb
    )

    cp = jnp.dot(v2_ref[...], Sw_ref[...], preferred_element_type=jnp.float32)
    ones = jnp.ones((_BN, 1), jnp.float32)
    center1 = jnp.concatenate([p_ref[...] + cp, ones], axis=1)
    center_ref[...] = center1

    b = batch_ref[0, 0, :]
    gids = lax.broadcasted_iota(jnp.int32, (_G, _BN), 0)
    onehot_t = (gids == b[None, :]).astype(jnp.float32)
    part = jnp.dot(onehot_t, center1, preferred_element_type=jnp.float32)

    @pl.when(blk == 0)
    def _():
        sums_ref[...] = jnp.zeros((_G, 4), jnp.float32)
        wcomb_ref[...] = jnp.dot(W0p_ref[...], Wbond_ref[...],
                                 preferred_element_type=jnp.float32)

    sums_ref[...] += part


def _nodeB_body(t_ref, center_ref, batch_ref, sums_ref,
                coords_ref, tA_ref, tB_ref):
    sums = sums_ref[...]
    mean = sums[:, :3] / jnp.maximum(sums[:, 3:4], 1.0)
    b = batch_ref[0, 0, :]
    gids = lax.broadcasted_iota(jnp.int32, (_G, _BN), 0)
    onehot_t = (gids == b[None, :]).astype(jnp.float32)
    meanb = lax.dot_general(onehot_t, mean, (((0,), (0,)), ((), ())),
                            preferred_element_type=jnp.float32)
    coords = center_ref[:, :3] - meanb
    coords_ref[...] = coords
    t = t_ref[...]
    zpad = jnp.zeros((_BN, _TDIM - _SDIM - 3), jnp.float32)
    tA_ref[...] = jnp.concatenate([t, coords, zpad], axis=1)
    tB_ref[...] = jnp.concatenate([t, -coords, zpad], axis=1)


def _node_stage(s, v2, p, batch, Ws, bs_row, Wa, ba_row, W0p, bb0_row, Wbond,
                bbond_row, Sw):
    full = lambda r, c: pl.BlockSpec((r, c), lambda b: (0, 0))
    atoms, t, center, sums, wcomb = pl.pallas_call(
        _nodeA_body,
        grid=(_NB,),
        in_specs=[
            pl.BlockSpec((_BN, _SDIM), lambda b: (b, 0)),
            pl.BlockSpec((_BN, 3 * _VDIM), lambda b: (b, 0)),
            pl.BlockSpec((_BN, 3), lambda b: (b, 0)),
            pl.BlockSpec((1, 1, _BN), lambda b: (b, 0, 0)),
            full(_SDIM, _SDIM),
            full(1, _SDIM),
            full(_NUM_ATOM, _SDIM),
            full(1, _NUM_ATOM),
            full(_SDIM, _SDIM),
            full(1, _SDIM),
            full(_SDIM, _EDGE_DIM),
            full(1, _SDIM),
            full(3 * _VDIM, 3),
        ],
        out_specs=[
            pl.BlockSpec((_BN, _NUM_ATOM), lambda b: (b, 0)),
            pl.BlockSpec((_BN, _SDIM), lambda b: (b, 0)),
            pl.BlockSpec((_BN, 4), lambda b: (b, 0)),
            full(_G, 4),
            full(_SDIM, _EDGE_DIM),
        ],
        out_shape=[
            jax.ShapeDtypeStruct((_N, _NUM_ATOM), jnp.float32),
            jax.ShapeDtypeStruct((_N, _SDIM), jnp.float32),
            jax.ShapeDtypeStruct((_N, 4), jnp.float32),
            jax.ShapeDtypeStruct((_G, 4), jnp.float32),
            jax.ShapeDtypeStruct((_SDIM, _EDGE_DIM), jnp.float32),
        ],
    )(s, v2, p, batch, Ws, bs_row, Wa, ba_row, W0p, bb0_row, Wbond,
      bbond_row, Sw)

    coords, tA, tB = pl.pallas_call(
        _nodeB_body,
        grid=(_NB,),
        in_specs=[
            pl.BlockSpec((_BN, _SDIM), lambda b: (b, 0)),
            pl.BlockSpec((_BN, 4), lambda b: (b, 0)),
            pl.BlockSpec((1, 1, _BN), lambda b: (b, 0, 0)),
            full(_G, 4),
        ],
        out_specs=[
            pl.BlockSpec((_BN, 3), lambda b: (b, 0)),
            pl.BlockSpec((_BN, _TDIM), lambda b: (b, 0)),
            pl.BlockSpec((_BN, _TDIM), lambda b: (b, 0)),
        ],
        out_shape=[
            jax.ShapeDtypeStruct((_N, 3), jnp.float32),
            jax.ShapeDtypeStruct((_N, _TDIM), jnp.float32),
            jax.ShapeDtypeStruct((_N, _TDIM), jnp.float32),
        ],
    )(t, center, batch, sums)

    return coords, atoms, tA, tB, wcomb


def _sc_body(tA_hbm, tB_hbm, idxi_hbm, idxj_hbm, g_hbm,
             idxi_v, idxj_v, rows_v, sem_g, sem_o):
    cid = lax.axis_index("c")
    sid = lax.axis_index("s")
    wid = sid * _NC + cid
    base = wid * _ET

    pltpu.sync_copy(idxi_hbm.at[wid], idxi_v)
    pltpu.sync_copy(idxj_hbm.at[wid], idxj_v)

    def chunk(c, _):
        pltpu.async_copy(tA_hbm.at[idxi_v.at[c]], rows_v, sem_g).wait()
        pltpu.async_copy(tB_hbm.at[idxj_v.at[c]], rows_v, sem_g,
                         add=True).wait()
        pltpu.async_copy(rows_v, g_hbm.at[pl.ds(base + c * _CH, _CH)],
                         sem_o).wait()
        return 0

    lax.fori_loop(0, _NFULL, chunk, 0)

    c = _NFULL
    pltpu.async_copy(tA_hbm.at[idxi_v.at[c]], rows_v, sem_g).wait()
    pltpu.async_copy(tB_hbm.at[idxj_v.at[c]], rows_v, sem_g, add=True).wait()
    pltpu.async_copy(rows_v.at[pl.ds(0, _REM)],
                     g_hbm.at[pl.ds(base + c * _CH, _REM)], sem_o).wait()


def _edge_gather_sc(tA, tB, idxi_pad, idxj_pad):
    mesh = plsc.VectorSubcoreMesh(core_axis_name="c", subcore_axis_name="s")
    f = pl.kernel(
        _sc_body,
        out_type=jax.ShapeDtypeStruct((_E, _TDIM), jnp.float32),
        mesh=mesh,
        scratch_types=[
            pltpu.VMEM((_NCHUNK, _CH), jnp.int32),
            pltpu.VMEM((_NCHUNK, _CH), jnp.int32),
            pltpu.VMEM((_CH, _TDIM), jnp.float32),
            pltpu.SemaphoreType.DMA,
            pltpu.SemaphoreType.DMA,
        ],
        compiler_params=pltpu.CompilerParams(use_tc_tiling_on_sc=False),
    )
    return f(tA, tB, idxi_pad, idxj_pad)


def _mlp_body(g_ref, e_ref, wcomb_ref, wd_ref, W1_ref, b1_ref, out_ref):
    gd = g_ref[:, _SDIM:]
    d2 = jnp.sum(gd * gd, axis=1, keepdims=True)
    d = jnp.sqrt(d2)
    h = (g_ref[:, :_SDIM]
         + jnp.dot(e_ref[...], wcomb_ref[...].T,
                   preferred_element_type=jnp.float32)
         + d * wd_ref[...])
    h = h * jax.nn.sigmoid(h)
    out_ref[...] = (
        jnp.dot(h, W1_ref[...].T, preferred_element_type=jnp.float32)
        + b1_ref[...]
    )


def _edge_mlp(g, e, wcomb, wd_row, W1, b1_row):
    nblk = _E // _BLK
    return pl.pallas_call(
        _mlp_body,
        grid=(nblk,),
        in_specs=[
            pl.BlockSpec((_BLK, _TDIM), lambda b: (b, 0)),
            pl.BlockSpec((_BLK, _EDGE_DIM), lambda b: (b, 0)),
            pl.BlockSpec((_SDIM, _EDGE_DIM), lambda b: (0, 0)),
            pl.BlockSpec((1, _SDIM), lambda b: (0, 0)),
            pl.BlockSpec((_NUM_BOND, _SDIM), lambda b: (0, 0)),
            pl.BlockSpec((1, _NUM_BOND), lambda b: (0, 0)),
        ],
        out_specs=pl.BlockSpec((_BLK, _NUM_BOND), lambda b: (b, 0)),
        out_shape=jax.ShapeDtypeStruct((_E, _NUM_BOND), jnp.float32),
    )(g, e, wcomb, wd_row, W1, b1_row)


def kernel(s, v, p, e, batch, edge_index_global, W_shared, b_shared, W_bond,
           b_bond, W_b0, b_b0, W_b1, b_b1, W_coords, W_atoms, b_atoms):
    W0p = W_b0[:, :_SDIM]
    wd_row = W_b0[:, _SDIM].reshape(1, _SDIM)
    bs_row = b_shared.reshape(1, _SDIM)
    ba_row = b_atoms.reshape(1, _NUM_ATOM)
    bb0_row = b_b0.reshape(1, _SDIM)
    bbond_row = b_bond.reshape(1, _SDIM)
    b1_row = b_b1.reshape(1, _NUM_BOND)

    lidx = jnp.arange(3 * _VDIM)
    Sw = jnp.where((lidx[:, None] // _VDIM) == jnp.arange(3)[None, :],
                   jnp.tile(W_coords.reshape(-1), 3)[:, None], 0.0)
    Sw = Sw.astype(jnp.float32)
    v2 = v.reshape(_N, 3 * _VDIM)
    batch3 = batch.reshape(_NB, 1, _BN)

    coords_pred, atoms_pred, tA, tB, wcomb = _node_stage(
        s, v2, p, batch3, W_shared, bs_row, W_atoms, ba_row, W0p, bb0_row,
        W_bond, bbond_row, Sw)

    idx_j = edge_index_global[0]
    idx_i = edge_index_global[1]
    pad = _NCHUNK * _CH - _ET
    idxi_pad = jnp.pad(idx_i.reshape(_NW, _ET), ((0, 0), (0, pad)))
    idxi_pad = idxi_pad.reshape(_NW, _NCHUNK, _CH)
    idxj_pad = jnp.pad(idx_j.reshape(_NW, _ET), ((0, 0), (0, pad)))
    idxj_pad = idxj_pad.reshape(_NW, _NCHUNK, _CH)

    g = _edge_gather_sc(tA, tB, idxi_pad, idxj_pad)

    bonds_pred = _edge_mlp(g, e, wcomb, wd_row, W_b1, b1_row)

    return (coords_pred, atoms_pred, bonds_pred)


# speedup vs baseline: 2.7595x; 1.0532x over previous
import functools

import jax
import jax.numpy as jnp
from jax import lax
from jax.experimental import pallas as pl
from jax.experimental.pallas import tpu as pltpu
from jax.experimental.pallas import tpu_sc as plsc

_N = 10000
_E = 320000
_SDIM = 128
_VDIM = 64
_EDGE_DIM = 16
_NUM_ATOM = 16
_NUM_BOND = 5
_G = 256

_NC = 2
_NS = 16
_NW = _NC * _NS
_ET = _E // _NW
_CH = 128
_NFULL = _ET // _CH
_REM = _ET - _NFULL * _CH
_NCHUNK = _NFULL + 1

_TDIM = 144
_BLK = 2560
_BN = 2000
_NB = _N // _BN


def _nodeA_body(s_ref, v2_ref, p_ref, batch_ref, Ws_ref, bs_ref, Wa_ref,
                ba_ref, W0p_ref, bb0_ref, Wbond_ref, bbond_ref, Sw_ref,
                atoms_ref, t_ref, center_ref, sums_ref, wcomb_ref):
    blk = pl.program_id(0)
    s2 = jnp.dot(s_ref[...], Ws_ref[...].T, preferred_element_type=jnp.float32)
    s2 = s2 + bs_ref[...]
    s2 = s2 * jax.nn.sigmoid(s2)

    atoms_ref[...] = (
        jnp.dot(s2, Wa_ref[...].T, preferred_element_type=jnp.float32)
        + ba_ref[...]
    )

    tb = 0.5 * (bb0_ref[...]
                + jnp.dot(bbond_ref[...], W0p_ref[...].T,
                          preferred_element_type=jnp.float32))
    t_ref[...] = (
        jnp.dot(s2, W0p_ref[...].T, preferred_element_type=jnp.float32) + tb
    )

    cp = jnp.dot(v2_ref[...], Sw_ref[...], preferred_element_type=jnp.float32)
    ones = jnp.ones((_BN, 1), jnp.float32)
    center1 = jnp.concatenate([p_ref[...] + cp, ones], axis=1)
    center_ref[...] = center1

    b = batch_ref[0, 0, :]
    gids = lax.broadcasted_iota(jnp.int32, (_G, _BN), 0)
    onehot_t = (gids == b[None, :]).astype(jnp.float32)
    part = jnp.dot(onehot_t, center1, preferred_element_type=jnp.float32)

    @pl.when(blk == 0)
    def _():
        sums_ref[...] = jnp.zeros((_G, 4), jnp.float32)
        wcomb_ref[...] = jnp.dot(W0p_ref[...], Wbond_ref[...],
                                 preferred_element_type=jnp.float32)

    sums_ref[...] += part


def _nodeB_body(t_ref, center_ref, batch_ref, sums_ref,
                coords_ref, tA_ref, tB_ref):
    sums = sums_ref[...]
    mean = sums[:, :3] / jnp.maximum(sums[:, 3:4], 1.0)
    b = batch_ref[0, 0, :]
    gids = lax.broadcasted_iota(jnp.int32, (_G, _BN), 0)
    onehot_t = (gids == b[None, :]).astype(jnp.float32)
    meanb = lax.dot_general(onehot_t, mean, (((0,), (0,)), ((), ())),
                            preferred_element_type=jnp.float32)
    coords = center_ref[:, :3] - meanb
    coords_ref[...] = coords
    t = t_ref[...]
    zpad = jnp.zeros((_BN, _TDIM - _SDIM - 3), jnp.float32)
    tA_ref[...] = jnp.concatenate([t, coords, zpad], axis=1)
    tB_ref[...] = jnp.concatenate([t, -coords, zpad], axis=1)


def _node_stage(s, v2, p, batch, Ws, bs_row, Wa, ba_row, W0p, bb0_row, Wbond,
                bbond_row, Sw):
    full = lambda r, c: pl.BlockSpec((r, c), lambda b: (0, 0))
    atoms, t, center, sums, wcomb = pl.pallas_call(
        _nodeA_body,
        grid=(_NB,),
        in_specs=[
            pl.BlockSpec((_BN, _SDIM), lambda b: (b, 0)),
            pl.BlockSpec((_BN, 3 * _VDIM), lambda b: (b, 0)),
            pl.BlockSpec((_BN, 3), lambda b: (b, 0)),
            pl.BlockSpec((1, 1, _BN), lambda b: (b, 0, 0)),
            full(_SDIM, _SDIM),
            full(1, _SDIM),
            full(_NUM_ATOM, _SDIM),
            full(1, _NUM_ATOM),
            full(_SDIM, _SDIM),
            full(1, _SDIM),
            full(_SDIM, _EDGE_DIM),
            full(1, _SDIM),
            full(3 * _VDIM, 3),
        ],
        out_specs=[
            pl.BlockSpec((_BN, _NUM_ATOM), lambda b: (b, 0)),
            pl.BlockSpec((_BN, _SDIM), lambda b: (b, 0)),
            pl.BlockSpec((_BN, 4), lambda b: (b, 0)),
            full(_G, 4),
            full(_SDIM, _EDGE_DIM),
        ],
        out_shape=[
            jax.ShapeDtypeStruct((_N, _NUM_ATOM), jnp.float32),
            jax.ShapeDtypeStruct((_N, _SDIM), jnp.float32),
            jax.ShapeDtypeStruct((_N, 4), jnp.float32),
            jax.ShapeDtypeStruct((_G, 4), jnp.float32),
            jax.ShapeDtypeStruct((_SDIM, _EDGE_DIM), jnp.float32),
        ],
    )(s, v2, p, batch, Ws, bs_row, Wa, ba_row, W0p, bb0_row, Wbond,
      bbond_row, Sw)

    coords, tA, tB = pl.pallas_call(
        _nodeB_body,
        grid=(_NB,),
        in_specs=[
            pl.BlockSpec((_BN, _SDIM), lambda b: (b, 0)),
            pl.BlockSpec((_BN, 4), lambda b: (b, 0)),
            pl.BlockSpec((1, 1, _BN), lambda b: (b, 0, 0)),
            full(_G, 4),
        ],
        out_specs=[
            pl.BlockSpec((_BN, 3), lambda b: (b, 0)),
            pl.BlockSpec((_BN, _TDIM), lambda b: (b, 0)),
            pl.BlockSpec((_BN, _TDIM), lambda b: (b, 0)),
        ],
        out_shape=[
            jax.ShapeDtypeStruct((_N, 3), jnp.float32),
            jax.ShapeDtypeStruct((_N, _TDIM), jnp.float32),
            jax.ShapeDtypeStruct((_N, _TDIM), jnp.float32),
        ],
    )(t, center, batch, sums)

    return coords, atoms, tA, tB, wcomb


_NBUF = 3


def _sc_body(tA_hbm, tB_hbm, idxi_hbm, idxj_hbm, g_hbm,
             idxi_v, idxj_v, rows_v, sem0, sem1, sem2):
    sems = (sem0, sem1, sem2)
    cid = lax.axis_index("c")
    sid = lax.axis_index("s")
    wid = sid * _NC + cid
    base = wid * _ET

    pltpu.sync_copy(idxi_hbm.at[wid], idxi_v)
    pltpu.sync_copy(idxj_hbm.at[wid], idxj_v)

    def mkA(c):
        return pltpu.async_copy(tA_hbm.at[idxi_v.at[c]], rows_v.at[c % _NBUF],
                                sems[c % _NBUF])

    def mkB(c):
        return pltpu.async_copy(tB_hbm.at[idxj_v.at[c]], rows_v.at[c % _NBUF],
                                sems[c % _NBUF], add=True)

    def mkW(c):
        nrows = _REM if c == _NFULL else _CH
        return pltpu.async_copy(rows_v.at[c % _NBUF, pl.ds(0, nrows)],
                                g_hbm.at[pl.ds(base + c * _CH, nrows)],
                                sems[c % _NBUF])

    descA = [None] * _NCHUNK
    descB = [None] * _NCHUNK
    descW = [None] * _NCHUNK
    for step in range(_NCHUNK + 2):
        cW = step - 2
        cB = step - 1
        cA = step
        if cW >= 0:
            descB[cW].wait()
            descW[cW] = mkW(cW)
        if 0 <= cB < _NCHUNK:
            descA[cB].wait()
            descB[cB] = mkB(cB)
        if cA < _NCHUNK:
            if cA >= _NBUF:
                descW[cA - _NBUF].wait()
            descA[cA] = mkA(cA)
    for c in range(max(0, _NCHUNK - _NBUF), _NCHUNK):
        descW[c].wait()


def _edge_gather_sc(tA, tB, idxi_pad, idxj_pad):
    mesh = plsc.VectorSubcoreMesh(core_axis_name="c", subcore_axis_name="s")
    f = pl.kernel(
        _sc_body,
        out_type=jax.ShapeDtypeStruct((_E, _TDIM), jnp.float32),
        mesh=mesh,
        scratch_types=[
            pltpu.VMEM((_NCHUNK, _CH), jnp.int32),
            pltpu.VMEM((_NCHUNK, _CH), jnp.int32),
            pltpu.VMEM((_NBUF, _CH, _TDIM), jnp.float32),
            pltpu.SemaphoreType.DMA,
            pltpu.SemaphoreType.DMA,
            pltpu.SemaphoreType.DMA,
        ],
        compiler_params=pltpu.CompilerParams(use_tc_tiling_on_sc=False),
    )
    return f(tA, tB, idxi_pad, idxj_pad)


def _mlp_body(g_ref, e_ref, wcomb_ref, wd_ref, W1_ref, b1_ref, out_ref):
    gd = g_ref[:, _SDIM:]
    d2 = jnp.sum(gd * gd, axis=1, keepdims=True)
    d = jnp.sqrt(d2)
    h = (g_ref[:, :_SDIM]
         + jnp.dot(e_ref[...], wcomb_ref[...].T,
                   preferred_element_type=jnp.float32)
         + d * wd_ref[...])
    h = h * jax.nn.sigmoid(h)
    out_ref[...] = (
        jnp.dot(h, W1_ref[...].T, preferred_element_type=jnp.float32)
        + b1_ref[...]
    )


def _edge_mlp(g, e, wcomb, wd_row, W1, b1_row):
    nblk = _E // _BLK
    return pl.pallas_call(
        _mlp_body,
        grid=(nblk,),
        in_specs=[
            pl.BlockSpec((_BLK, _TDIM), lambda b: (b, 0)),
            pl.BlockSpec((_BLK, _EDGE_DIM), lambda b: (b, 0)),
            pl.BlockSpec((_SDIM, _EDGE_DIM), lambda b: (0, 0)),
            pl.BlockSpec((1, _SDIM), lambda b: (0, 0)),
            pl.BlockSpec((_NUM_BOND, _SDIM), lambda b: (0, 0)),
            pl.BlockSpec((1, _NUM_BOND), lambda b: (0, 0)),
        ],
        out_specs=pl.BlockSpec((_BLK, _NUM_BOND), lambda b: (b, 0)),
        out_shape=jax.ShapeDtypeStruct((_E, _NUM_BOND), jnp.float32),
    )(g, e, wcomb, wd_row, W1, b1_row)


def kernel(s, v, p, e, batch, edge_index_global, W_shared, b_shared, W_bond,
           b_bond, W_b0, b_b0, W_b1, b_b1, W_coords, W_atoms, b_atoms):
    W0p = W_b0[:, :_SDIM]
    wd_row = W_b0[:, _SDIM].reshape(1, _SDIM)
    bs_row = b_shared.reshape(1, _SDIM)
    ba_row = b_atoms.reshape(1, _NUM_ATOM)
    bb0_row = b_b0.reshape(1, _SDIM)
    bbond_row = b_bond.reshape(1, _SDIM)
    b1_row = b_b1.reshape(1, _NUM_BOND)

    lidx = jnp.arange(3 * _VDIM)
    Sw = jnp.where((lidx[:, None] // _VDIM) == jnp.arange(3)[None, :],
                   jnp.tile(W_coords.reshape(-1), 3)[:, None], 0.0)
    Sw = Sw.astype(jnp.float32)
    v2 = v.reshape(_N, 3 * _VDIM)
    batch3 = batch.reshape(_NB, 1, _BN)

    coords_pred, atoms_pred, tA, tB, wcomb = _node_stage(
        s, v2, p, batch3, W_shared, bs_row, W_atoms, ba_row, W0p, bb0_row,
        W_bond, bbond_row, Sw)

    idx_j = edge_index_global[0]
    idx_i = edge_index_global[1]
    pad = _NCHUNK * _CH - _ET
    idxi_pad = jnp.pad(idx_i.reshape(_NW, _ET), ((0, 0), (0, pad)))
    idxi_pad = idxi_pad.reshape(_NW, _NCHUNK, _CH)
    idxj_pad = jnp.pad(idx_j.reshape(_NW, _ET), ((0, 0), (0, pad)))
    idxj_pad = idxj_pad.reshape(_NW, _NCHUNK, _CH)

    g = _edge_gather_sc(tA, tB, idxi_pad, idxj_pad)

    bonds_pred = _edge_mlp(g, e, wcomb, wd_row, W_b1, b1_row)

    return (coords_pred, atoms_pred, bonds_pred)


# speedup vs baseline: 3.4915x; 1.2653x over previous
import functools

import jax
import jax.numpy as jnp
from jax import lax
from jax.experimental import pallas as pl
from jax.experimental.pallas import tpu as pltpu
from jax.experimental.pallas import tpu_sc as plsc

_N = 10000
_E = 320000
_SDIM = 128
_VDIM = 64
_EDGE_DIM = 16
_NUM_ATOM = 16
_NUM_BOND = 5
_G = 256

_NC = 2
_NS = 16
_NW = _NC * _NS
_S = 1
_ES = _E // _S
_ET = _ES // _NW
_CH = 128
_NFULL = _ET // _CH
_REM = _ET - _NFULL * _CH
_NCHUNK = _NFULL + 1

_TDIM = 144
_DDIM = 16
_BLK = 2560
_BN = 2000
_NB = _N // _BN


def _nodeA_body(s_ref, v2_ref, p_ref, batch_ref, Ws_ref, bs_ref, Wa_ref,
                ba_ref, W0p_ref, bb0_ref, Wbond_ref, bbond_ref, Sw_ref,
                atoms_ref, t_ref, center_ref, sums_ref, wcomb_ref):
    blk = pl.program_id(0)
    s2 = jnp.dot(s_ref[...], Ws_ref[...].T, preferred_element_type=jnp.float32)
    s2 = s2 + bs_ref[...]
    s2 = s2 * jax.nn.sigmoid(s2)

    atoms_ref[...] = (
        jnp.dot(s2, Wa_ref[...].T, preferred_element_type=jnp.float32)
        + ba_ref[...]
    )

    tb = 0.5 * (bb0_ref[...]
                + jnp.dot(bbond_ref[...], W0p_ref[...].T,
                          preferred_element_type=jnp.float32))
    t_ref[...] = (
        jnp.dot(s2, W0p_ref[...].T, preferred_element_type=jnp.float32) + tb
    )

    cp = jnp.dot(v2_ref[...], Sw_ref[...], preferred_element_type=jnp.float32)
    ones = jnp.ones((_BN, 1), jnp.float32)
    center1 = jnp.concatenate([p_ref[...] + cp, ones], axis=1)
    center_ref[...] = center1

    b = batch_ref[0, 0, :]
    gids = lax.broadcasted_iota(jnp.int32, (_G, _BN), 0)
    onehot_t = (gids == b[None, :]).astype(jnp.float32)
    part = jnp.dot(onehot_t, center1, preferred_element_type=jnp.float32)

    @pl.when(blk == 0)
    def _():
        sums_ref[...] = jnp.zeros((_G, 4), jnp.float32)
        wcomb_ref[...] = jnp.dot(W0p_ref[...], Wbond_ref[...],
                                 preferred_element_type=jnp.float32)

    sums_ref[...] += part


def _nodeB_body(t_ref, center_ref, batch_ref, sums_ref,
                coords_ref, tA_ref, tB_ref):
    sums = sums_ref[...]
    mean = sums[:, :3] / jnp.maximum(sums[:, 3:4], 1.0)
    b = batch_ref[0, 0, :]
    gids = lax.broadcasted_iota(jnp.int32, (_G, _BN), 0)
    onehot_t = (gids == b[None, :]).astype(jnp.float32)
    meanb = lax.dot_general(onehot_t, mean, (((0,), (0,)), ((), ())),
                            preferred_element_type=jnp.float32)
    coords = center_ref[:, :3] - meanb
    coords_ref[...] = coords
    t = t_ref[...]
    zpad = jnp.zeros((_BN, _TDIM - _SDIM - 3), jnp.float32)
    tA_ref[...] = jnp.concatenate([t, coords, zpad], axis=1)
    tB_ref[...] = jnp.concatenate([t, -coords, zpad], axis=1)


def _node_stage(s, v2, p, batch, Ws, bs_row, Wa, ba_row, W0p, bb0_row, Wbond,
                bbond_row, Sw):
    full = lambda r, c: pl.BlockSpec((r, c), lambda b: (0, 0))
    atoms, t, center, sums, wcomb = pl.pallas_call(
        _nodeA_body,
        grid=(_NB,),
        in_specs=[
            pl.BlockSpec((_BN, _SDIM), lambda b: (b, 0)),
            pl.BlockSpec((_BN, 3 * _VDIM), lambda b: (b, 0)),
            pl.BlockSpec((_BN, 3), lambda b: (b, 0)),
            pl.BlockSpec((1, 1, _BN), lambda b: (b, 0, 0)),
            full(_SDIM, _SDIM),
            full(1, _SDIM),
            full(_NUM_ATOM, _SDIM),
            full(1, _NUM_ATOM),
            full(_SDIM, _SDIM),
            full(1, _SDIM),
            full(_SDIM, _EDGE_DIM),
            full(1, _SDIM),
            full(3 * _VDIM, 3),
        ],
        out_specs=[
            pl.BlockSpec((_BN, _NUM_ATOM), lambda b: (b, 0)),
            pl.BlockSpec((_BN, _SDIM), lambda b: (b, 0)),
            pl.BlockSpec((_BN, 4), lambda b: (b, 0)),
            full(_G, 4),
            full(_SDIM, _EDGE_DIM),
        ],
        out_shape=[
            jax.ShapeDtypeStruct((_N, _NUM_ATOM), jnp.float32),
            jax.ShapeDtypeStruct((_N, _SDIM), jnp.float32),
            jax.ShapeDtypeStruct((_N, 4), jnp.float32),
            jax.ShapeDtypeStruct((_G, 4), jnp.float32),
            jax.ShapeDtypeStruct((_SDIM, _EDGE_DIM), jnp.float32),
        ],
    )(s, v2, p, batch, Ws, bs_row, Wa, ba_row, W0p, bb0_row, Wbond,
      bbond_row, Sw)

    coords, tA, tB = pl.pallas_call(
        _nodeB_body,
        grid=(_NB,),
        in_specs=[
            pl.BlockSpec((_BN, _SDIM), lambda b: (b, 0)),
            pl.BlockSpec((_BN, 4), lambda b: (b, 0)),
            pl.BlockSpec((1, 1, _BN), lambda b: (b, 0, 0)),
            full(_G, 4),
        ],
        out_specs=[
            pl.BlockSpec((_BN, 3), lambda b: (b, 0)),
            pl.BlockSpec((_BN, _TDIM), lambda b: (b, 0)),
            pl.BlockSpec((_BN, _TDIM), lambda b: (b, 0)),
        ],
        out_shape=[
            jax.ShapeDtypeStruct((_N, 3), jnp.float32),
            jax.ShapeDtypeStruct((_N, _TDIM), jnp.float32),
            jax.ShapeDtypeStruct((_N, _TDIM), jnp.float32),
        ],
    )(t, center, batch, sums)

    return coords, atoms, tA, tB, wcomb


_NBUF = 4


def _sc_body(tA_hbm, tB_hbm, idxi_hbm, idxj_hbm, g_hbm, gd_hbm,
             idxi_v, idxj_v, rows_v, sem0, sem1, sem2, sem3,
             semd0, semd1, semd2, semd3):
    sems = (sem0, sem1, sem2, sem3)
    semsd = (semd0, semd1, semd2, semd3)
    cid = lax.axis_index("c")
    sid = lax.axis_index("s")
    wid = sid * _NC + cid
    base = wid * _ET

    pltpu.sync_copy(idxi_hbm.at[wid], idxi_v)
    pltpu.sync_copy(idxj_hbm.at[wid], idxj_v)

    def mkA(c):
        return pltpu.async_copy(tA_hbm.at[idxi_v.at[c]], rows_v.at[c % _NBUF],
                                sems[c % _NBUF])

    def mkB(c):
        return pltpu.async_copy(tB_hbm.at[idxj_v.at[c]], rows_v.at[c % _NBUF],
                                sems[c % _NBUF], add=True)

    def mkW(c):
        nrows = _REM if c == _NFULL else _CH
        return pltpu.async_copy(
            rows_v.at[c % _NBUF, pl.ds(0, nrows), pl.ds(0, _SDIM)],
            g_hbm.at[pl.ds(base + c * _CH, nrows)], sems[c % _NBUF])

    def mkWd(c):
        nrows = _REM if c == _NFULL else _CH
        return pltpu.async_copy(
            rows_v.at[c % _NBUF, pl.ds(0, nrows), pl.ds(_SDIM, _DDIM)],
            gd_hbm.at[pl.ds(base + c * _CH, nrows)], semsd[c % _NBUF])

    descA = [None] * _NCHUNK
    descB = [None] * _NCHUNK
    descW = [None] * _NCHUNK
    descWd = [None] * _NCHUNK
    for step in range(_NCHUNK + 2):
        cW = step - 2
        cB = step - 1
        cA = step
        if cW >= 0:
            descB[cW].wait()
            descW[cW] = mkW(cW)
            descWd[cW] = mkWd(cW)
        if 0 <= cB < _NCHUNK:
            descA[cB].wait()
            descB[cB] = mkB(cB)
        if cA < _NCHUNK:
            if cA >= _NBUF:
                descW[cA - _NBUF].wait()
                descWd[cA - _NBUF].wait()
            descA[cA] = mkA(cA)
    for c in range(max(0, _NCHUNK - _NBUF), _NCHUNK):
        descW[c].wait()
        descWd[c].wait()


def _edge_gather_sc(tA, tB, idxi_pad, idxj_pad):
    mesh = plsc.VectorSubcoreMesh(core_axis_name="c", subcore_axis_name="s")
    f = pl.kernel(
        _sc_body,
        out_type=[
            jax.ShapeDtypeStruct((_ES, _SDIM), jnp.float32),
            jax.ShapeDtypeStruct((_ES, _DDIM), jnp.float32),
        ],
        mesh=mesh,
        scratch_types=[
            pltpu.VMEM((_NCHUNK, _CH), jnp.int32),
            pltpu.VMEM((_NCHUNK, _CH), jnp.int32),
            pltpu.VMEM((_NBUF, _CH, _TDIM), jnp.float32),
            pltpu.SemaphoreType.DMA,
            pltpu.SemaphoreType.DMA,
            pltpu.SemaphoreType.DMA,
            pltpu.SemaphoreType.DMA,
            pltpu.SemaphoreType.DMA,
            pltpu.SemaphoreType.DMA,
            pltpu.SemaphoreType.DMA,
            pltpu.SemaphoreType.DMA,
        ],
        compiler_params=pltpu.CompilerParams(use_tc_tiling_on_sc=False),
    )
    return f(tA, tB, idxi_pad, idxj_pad)


def _mlp_body(g_ref, gd_ref, e_ref, wcomb_ref, wd_ref, W1_ref, b1_ref,
              out_ref):
    sq = gd_ref[...]
    sq = sq * sq
    lm = lax.broadcasted_iota(jnp.int32, (_SDIM, 8), 0) // _DDIM
    M = (lm == lax.broadcasted_iota(jnp.int32, (_SDIM, 8), 1))
    d2g = jnp.dot(sq, M.astype(jnp.float32),
                  preferred_element_type=jnp.float32)
    dg = jnp.sqrt(d2g)
    nr = lax.broadcasted_iota(jnp.int32, (_BLK, _BLK // 8), 0) // 8
    P = (nr == lax.broadcasted_iota(jnp.int32, (_BLK, _BLK // 8), 1))
    A = jnp.dot(P.astype(jnp.float32), dg,
                preferred_element_type=jnp.float32)
    nm = lax.broadcasted_iota(jnp.int32, (_BLK, 8), 0) % 8
    sel = (nm == lax.broadcasted_iota(jnp.int32, (_BLK, 8), 1))
    d = jnp.dot(jnp.where(sel, A, 0.0), jnp.ones((8, 1), jnp.float32),
                preferred_element_type=jnp.float32)

    h = (g_ref[...]
         + jnp.dot(e_ref[...], wcomb_ref[...].T,
                   preferred_element_type=jnp.float32)
         + d * wd_ref[...])
    h = h * jax.nn.sigmoid(h)
    out_ref[...] = (
        jnp.dot(h, W1_ref[...].T, preferred_element_type=jnp.float32)
        + b1_ref[...]
    )


def _edge_mlp(g, gdp, e, wcomb, wd_row, W1, b1_row):
    nblk = _ES // _BLK
    return pl.pallas_call(
        _mlp_body,
        grid=(nblk,),
        in_specs=[
            pl.BlockSpec((_BLK, _SDIM), lambda b: (b, 0)),
            pl.BlockSpec((_BLK // 8, _SDIM), lambda b: (b, 0)),
            pl.BlockSpec((_BLK, _EDGE_DIM), lambda b: (b, 0)),
            pl.BlockSpec((_SDIM, _EDGE_DIM), lambda b: (0, 0)),
            pl.BlockSpec((1, _SDIM), lambda b: (0, 0)),
            pl.BlockSpec((_NUM_BOND, _SDIM), lambda b: (0, 0)),
            pl.BlockSpec((1, _NUM_BOND), lambda b: (0, 0)),
        ],
        out_specs=pl.BlockSpec((_BLK, _NUM_BOND), lambda b: (b, 0)),
        out_shape=jax.ShapeDtypeStruct((_ES, _NUM_BOND), jnp.float32),
    )(g, gdp, e, wcomb, wd_row, W1, b1_row)


def kernel(s, v, p, e, batch, edge_index_global, W_shared, b_shared, W_bond,
           b_bond, W_b0, b_b0, W_b1, b_b1, W_coords, W_atoms, b_atoms):
    W0p = W_b0[:, :_SDIM]
    wd_row = W_b0[:, _SDIM].reshape(1, _SDIM)
    bs_row = b_shared.reshape(1, _SDIM)
    ba_row = b_atoms.reshape(1, _NUM_ATOM)
    bb0_row = b_b0.reshape(1, _SDIM)
    bbond_row = b_bond.reshape(1, _SDIM)
    b1_row = b_b1.reshape(1, _NUM_BOND)

    lidx = jnp.arange(3 * _VDIM)
    Sw = jnp.where((lidx[:, None] // _VDIM) == jnp.arange(3)[None, :],
                   jnp.tile(W_coords.reshape(-1), 3)[:, None], 0.0)
    Sw = Sw.astype(jnp.float32)
    v2 = v.reshape(_N, 3 * _VDIM)
    batch3 = batch.reshape(_NB, 1, _BN)

    coords_pred, atoms_pred, tA, tB, wcomb = _node_stage(
        s, v2, p, batch3, W_shared, bs_row, W_atoms, ba_row, W0p, bb0_row,
        W_bond, bbond_row, Sw)

    idx_j = edge_index_global[0]
    idx_i = edge_index_global[1]
    pad = _NCHUNK * _CH - _ET
    idxi_pad = jnp.pad(idx_i.reshape(_S, _NW, _ET), ((0, 0), (0, 0), (0, pad)))
    idxi_pad = idxi_pad.reshape(_S, _NW, _NCHUNK, _CH)
    idxj_pad = jnp.pad(idx_j.reshape(_S, _NW, _ET), ((0, 0), (0, 0), (0, pad)))
    idxj_pad = idxj_pad.reshape(_S, _NW, _NCHUNK, _CH)

    bonds = []
    for s in range(_S):
        g, gd = _edge_gather_sc(tA, tB, idxi_pad[s], idxj_pad[s])
        gdp = gd.reshape(_ES // 8, _SDIM)
        e_s = lax.slice_in_dim(e, s * _ES, (s + 1) * _ES, axis=0)
        bonds.append(_edge_mlp(g, gdp, e_s, wcomb, wd_row, W_b1, b1_row))
    bonds_pred = jnp.concatenate(bonds, axis=0)

    return (coords_pred, atoms_pred, bonds_pred)
